# baseline (speedup 1.0000x reference)
import functools

import jax
import jax.numpy as jnp
from jax import lax
from jax.experimental import pallas as pl
from jax.experimental.pallas import tpu as pltpu
from jax.experimental.pallas import tpu_sc as plsc

B = 4096
F_CAT = 26
F_NUM = 13
V = 100000
D = 32
F_TOT = F_CAT + F_NUM

CLO = (0, 33408, 66816)
CLEN = (33408, 33408, V - 66816)
NVEC = B // 16


def _sc_embed(tab_t, cat_t, ct_t, wb_flat):
    info = plsc.get_sparse_core_info()
    NC, NS = info.num_cores, info.num_subcores
    NW = NC * NS
    cat_pw = (F_CAT * D) // NW
    num_pw = (F_NUM * D) // NW
    mesh = plsc.VectorSubcoreMesh(core_axis_name="c", subcore_axis_name="s")

    @functools.partial(
        pl.kernel,
        mesh=mesh,
        compiler_params=pltpu.CompilerParams(
            use_tc_tiling_on_sc=True, needs_layout_passes=False),
        out_type=jax.ShapeDtypeStruct((F_TOT, D, B), jnp.float32),
        scratch_types=[
            pltpu.VMEM((CLEN[0],), jnp.float32),
            pltpu.VMEM((CLEN[1],), jnp.float32),
            pltpu.VMEM((CLEN[2],), jnp.float32),
            pltpu.VMEM((B,), jnp.int32),
            pltpu.VMEM((2 * (B + 128),), jnp.int32),
            pltpu.VMEM((B,), jnp.float32),
            pltpu.VMEM((2 * B,), jnp.float32),
            pltpu.VMEM((B,), jnp.float32),
            pltpu.VMEM((2 * F_NUM * D,), jnp.float32),
            pltpu.SemaphoreType.DMA,
            pltpu.SemaphoreType.DMA,
            pltpu.SemaphoreType.DMA,
            pltpu.SemaphoreType.DMA,
            pltpu.SemaphoreType.DMA,
        ],
    )
    def k(tab_hbm, cat_hbm, ct_hbm, wb_hbm, out_hbm,
          b0_v, b1_v, b2_v, raw_v, pk_v, ct_v, out_v, nout_v, wb_v,
          sem0, sem1, sem2, osem, nsem):
        wid = lax.axis_index("s") * NC + lax.axis_index("c")
        bufs = (b0_v, b1_v, b2_v)
        sems = (sem0, sem1, sem2)
        iota = lax.iota(jnp.int32, 16)
        pltpu.sync_copy(wb_hbm, wb_v)

        g0 = wid * cat_pw
        fa = g0 // D
        n0 = (fa + 1) * D - g0
        fb = jnp.minimum(fa + 1, F_CAT - 1)

        def plane_fd(p):
            g = g0 + p
            return g // D, g % D

        def fire(p, c):
            f, d = plane_fd(p)
            return pltpu.async_copy(
                tab_hbm.at[f, d, pl.ds(CLO[c], CLEN[c])], bufs[c], sems[c])

        def bucket(field, slot_off):
            pltpu.sync_copy(cat_hbm.at[field], raw_v)

            def count_body(i, carry):
                c0, c1 = carry
                for h in range(4):
                    vec = raw_v[pl.ds(i * 64 + h * 16, 16)]
                    m0 = vec < CLO[1]
                    m1 = jnp.logical_and(vec >= CLO[1], vec < CLO[2])
                    c0 = c0 + plsc.all_reduce_population_count(m0)[0]
                    c1 = c1 + plsc.all_reduce_population_count(m1)[0]
                return c0, c1

            c0, c1 = lax.fori_loop(
                0, NVEC // 4, count_body, (jnp.int32(0), jnp.int32(0)))
            s1, s2 = c0, c0 + c1

            def pack_body(i, carry):
                o0, o1, o2 = carry
                vec = raw_v[pl.ds(i * 16, 16)]
                pos = (iota + i * 16) << 16
                m0 = vec < CLO[1]
                m1 = jnp.logical_and(vec >= CLO[1], vec < CLO[2])
                m2 = vec >= CLO[2]
                plsc.store_compressed(
                    pk_v.at[pl.ds(slot_off + o0, 16)], vec | pos, mask=m0)
                plsc.store_compressed(
                    pk_v.at[pl.ds(slot_off + o1, 16)], (vec - CLO[1]) | pos, mask=m1)
                plsc.store_compressed(
                    pk_v.at[pl.ds(slot_off + o2, 16)], (vec - CLO[2]) | pos, mask=m2)
                o0 = o0 + plsc.all_reduce_population_count(m0)[0]
                o1 = o1 + plsc.all_reduce_population_count(m1)[0]
                o2 = o2 + plsc.all_reduce_population_count(m2)[0]
                return o0, o1, o2

            lax.fori_loop(0, NVEC, pack_body, (jnp.int32(0), s1, s2))
            return s1, s2

        pending = [fire(0, 0), fire(0, 1), fire(0, 2)]
        s1a, s2a = bucket(fa, 0)
        startsA = (jnp.int32(0), s1a, s2a, jnp.int32(B))
        startsB = None

        ocopies = [None, None]
        ncopy = [None]

        def num_plane(q):
            h = wid * num_pw + q
            j = h // D
            d = h % D
            if q == 0:
                pltpu.sync_copy(ct_hbm.at[j], ct_v)
            else:
                @pl.when(d == 0)
                def _():
                    pltpu.sync_copy(ct_hbm.at[j], ct_v)
            wsp = plsc.load_gather(
                wb_v, [jnp.full((16,), j * D + d, jnp.int32)])
            bsp = plsc.load_gather(
                wb_v, [jnp.full((16,), F_NUM * D + j * D + d, jnp.int32)])
            if ncopy[0] is not None:
                ncopy[0].wait()

            def num_body(i, carry, wsp=wsp, bsp=bsp):
                for h in range(4):
                    o = i * 64 + h * 16
                    cvec = ct_v[pl.ds(o, 16)]
                    nout_v[pl.ds(o, 16)] = jnp.maximum(cvec * wsp + bsp, 0.0)
                return carry

            lax.fori_loop(0, NVEC // 4, num_body, 0)
            ncopy[0] = pltpu.async_copy(
                nout_v, out_hbm.at[F_CAT + j, d], nsem)
        for p in range(cat_pw):
            f, d = plane_fd(p)
            in_b = jnp.int32(p) >= n0
            slot_off = in_b.astype(jnp.int32) * (B + 128)
            ob = p % 2
            if ocopies[ob] is not None:
                ocopies[ob].wait()
            for c in range(3):
                if p == 0:
                    st, en = startsA[c], startsA[c + 1]
                else:
                    st = jnp.where(in_b, startsB[c], startsA[c])
                    en = jnp.where(in_b, startsB[c + 1], startsA[c + 1])
                pending.pop(0).wait()
                buf = bufs[c]
                nv = (en - st + 63) // 64

                def cpass(i, carry, buf=buf, st=st, en=en,
                          slot_off=slot_off, ob=ob):
                    for h in range(4):
                        off = st + i * 64 + h * 16
                        w = pk_v[pl.ds(slot_off + off, 16)]
                        valid = (off + iota) < en
                        bidx = w & jnp.int32(0xFFFF)
                        pos = lax.shift_right_logical(w, 16) + ob * B
                        g = plsc.load_gather(buf, [bidx], mask=valid)
                        plsc.store_scatter(out_v, [pos], g, mask=valid)
                    return carry

                lax.fori_loop(0, nv, cpass, 0)
                nxt = 3 * p + c + 3
                if nxt < 3 * cat_pw:
                    pending.append(fire(nxt // 3, nxt % 3))
            ocopies[ob] = pltpu.async_copy(
                out_v.at[pl.ds(ob * B, B)], out_hbm.at[f, d], osem)
            if p == 0:
                s1b, s2b = bucket(fb, B + 128)
                startsB = (jnp.int32(0), s1b, s2b, jnp.int32(B))
            elif p % 2 == 1:
                num_plane(p // 2)
        for oc in ocopies:
            if oc is not None:
                oc.wait()

        if ncopy[0] is not None:
            ncopy[0].wait()

    return k(tab_t, cat_t, ct_t, wb_flat)


def kernel(continuous, categorical, tables, W_num, b_num):
    tab_t = tables.transpose(0, 2, 1)
    cat_t = categorical.T
    ct_t = continuous.T
    wb_flat = jnp.concatenate([W_num.reshape(-1), b_num.reshape(-1)])
    out = _sc_embed(tab_t, cat_t, ct_t, wb_flat)
    return out.transpose(2, 0, 1)


# speedup vs baseline: 1.0377x; 1.0377x over previous
import functools

import jax
import jax.numpy as jnp
from jax import lax
from jax.experimental import pallas as pl
from jax.experimental.pallas import tpu as pltpu
from jax.experimental.pallas import tpu_sc as plsc

B = 4096
F_CAT = 26
F_NUM = 13
V = 100000
D = 32
F_TOT = F_CAT + F_NUM

CLO = (0, 33408, 66816)
CLEN = (33408, 33408, V - 66816)
NVEC = B // 16


def _sc_embed(tab_t, cat_t, ct_t, wb_flat):
    info = plsc.get_sparse_core_info()
    NC, NS = info.num_cores, info.num_subcores
    NW = NC * NS
    cat_pw = (F_CAT * D) // NW
    num_pw = (F_NUM * D) // NW
    mesh = plsc.VectorSubcoreMesh(core_axis_name="c", subcore_axis_name="s")

    @functools.partial(
        pl.kernel,
        mesh=mesh,
        compiler_params=pltpu.CompilerParams(
            use_tc_tiling_on_sc=True, needs_layout_passes=False),
        out_type=jax.ShapeDtypeStruct((F_TOT, D, B), jnp.float32),
        scratch_types=[
            pltpu.VMEM((CLEN[0],), jnp.float32),
            pltpu.VMEM((CLEN[1],), jnp.float32),
            pltpu.VMEM((CLEN[2],), jnp.float32),
            pltpu.VMEM((B,), jnp.int32),
            pltpu.VMEM((2 * (B + 64),), jnp.int32),
            pltpu.VMEM((B,), jnp.float32),
            pltpu.VMEM((2 * B,), jnp.float32),
            pltpu.VMEM((B,), jnp.float32),
            pltpu.VMEM((2 * F_NUM * D,), jnp.float32),
            pltpu.SemaphoreType.DMA,
            pltpu.SemaphoreType.DMA,
            pltpu.SemaphoreType.DMA,
            pltpu.SemaphoreType.DMA,
            pltpu.SemaphoreType.DMA,
        ],
    )
    def k(tab_hbm, cat_hbm, ct_hbm, wb_hbm, out_hbm,
          b0_v, b1_v, b2_v, raw_v, pk_v, ct_v, out_v, nout_v, wb_v,
          sem0, sem1, sem2, osem, nsem):
        wid = lax.axis_index("s") * NC + lax.axis_index("c")
        bufs = (b0_v, b1_v, b2_v)
        sems = (sem0, sem1, sem2)
        iota = lax.iota(jnp.int32, 16)
        pltpu.sync_copy(wb_hbm, wb_v)

        g0 = wid * cat_pw
        fa = g0 // D
        n0 = (fa + 1) * D - g0
        fb = jnp.minimum(fa + 1, F_CAT - 1)

        def plane_fd(p):
            g = g0 + p
            return g // D, g % D

        def fire(p, c):
            f, d = plane_fd(p)
            return pltpu.async_copy(
                tab_hbm.at[f, d, pl.ds(CLO[c], CLEN[c])], bufs[c], sems[c])

        def bucket(field, slot_off):
            pltpu.sync_copy(cat_hbm.at[field], raw_v)

            def count_body(i, carry):
                c0, c1 = carry
                for h in range(4):
                    vec = raw_v[pl.ds(i * 64 + h * 16, 16)]
                    m0 = vec < CLO[1]
                    m1 = jnp.logical_and(vec >= CLO[1], vec < CLO[2])
                    c0 = c0 + plsc.all_reduce_population_count(m0)[0]
                    c1 = c1 + plsc.all_reduce_population_count(m1)[0]
                return c0, c1

            c0, c1 = lax.fori_loop(
                0, NVEC // 4, count_body, (jnp.int32(0), jnp.int32(0)))
            s1, s2 = c0, c0 + c1

            def pack_body(i, carry):
                o0, o1, o2 = carry
                vec = raw_v[pl.ds(i * 16, 16)]
                pos = (iota + i * 16) << 16
                m0 = vec < CLO[1]
                m1 = jnp.logical_and(vec >= CLO[1], vec < CLO[2])
                m2 = vec >= CLO[2]
                plsc.store_compressed(
                    pk_v.at[pl.ds(slot_off + o0, 16)], vec | pos, mask=m0)
                plsc.store_compressed(
                    pk_v.at[pl.ds(slot_off + o1, 16)], (vec - CLO[1]) | pos, mask=m1)
                plsc.store_compressed(
                    pk_v.at[pl.ds(slot_off + o2, 16)], (vec - CLO[2]) | pos, mask=m2)
                o0 = o0 + plsc.all_reduce_population_count(m0)[0]
                o1 = o1 + plsc.all_reduce_population_count(m1)[0]
                o2 = o2 + plsc.all_reduce_population_count(m2)[0]
                return o0, o1, o2

            lax.fori_loop(0, NVEC, pack_body, (jnp.int32(0), s1, s2))
            return s1, s2

        pending = [fire(0, 0), fire(0, 1), fire(0, 2)]
        s1a, s2a = bucket(fa, 0)
        startsA = (jnp.int32(0), s1a, s2a, jnp.int32(B))
        startsB = None

        ocopies = [None, None]
        ncopy = [None]

        def num_plane(q):
            h = wid * num_pw + q
            j = h // D
            d = h % D
            if q == 0:
                pltpu.sync_copy(ct_hbm.at[j], ct_v)
            else:
                @pl.when(d == 0)
                def _():
                    pltpu.sync_copy(ct_hbm.at[j], ct_v)
            wsp = plsc.load_gather(
                wb_v, [jnp.full((16,), j * D + d, jnp.int32)])
            bsp = plsc.load_gather(
                wb_v, [jnp.full((16,), F_NUM * D + j * D + d, jnp.int32)])
            if ncopy[0] is not None:
                ncopy[0].wait()

            def num_body(i, carry, wsp=wsp, bsp=bsp):
                for h in range(4):
                    o = i * 64 + h * 16
                    cvec = ct_v[pl.ds(o, 16)]
                    nout_v[pl.ds(o, 16)] = jnp.maximum(cvec * wsp + bsp, 0.0)
                return carry

            lax.fori_loop(0, NVEC // 4, num_body, 0)
            ncopy[0] = pltpu.async_copy(
                nout_v, out_hbm.at[F_CAT + j, d], nsem)
        for p in range(cat_pw):
            f, d = plane_fd(p)
            in_b = jnp.int32(p) >= n0
            slot_off = in_b.astype(jnp.int32) * (B + 64)
            ob = p % 2
            if ocopies[ob] is not None:
                ocopies[ob].wait()
            for c in range(3):
                if p == 0:
                    st, en = startsA[c], startsA[c + 1]
                else:
                    st = jnp.where(in_b, startsB[c], startsA[c])
                    en = jnp.where(in_b, startsB[c + 1], startsA[c + 1])
                pending.pop(0).wait()
                buf = bufs[c]
                nv = (en - st + 31) // 32

                def cpass(i, carry, buf=buf, st=st, en=en,
                          slot_off=slot_off, ob=ob):
                    for h in range(2):
                        off = st + i * 32 + h * 16
                        w = pk_v[pl.ds(slot_off + off, 16)]
                        valid = (off + iota) < en
                        bidx = w & jnp.int32(0xFFFF)
                        pos = lax.shift_right_logical(w, 16) + ob * B
                        g = plsc.load_gather(buf, [bidx], mask=valid)
                        plsc.store_scatter(out_v, [pos], g, mask=valid)
                    return carry

                lax.fori_loop(0, nv, cpass, 0)
                nxt = 3 * p + c + 3
                if nxt < 3 * cat_pw:
                    pending.append(fire(nxt // 3, nxt % 3))
            ocopies[ob] = pltpu.async_copy(
                out_v.at[pl.ds(ob * B, B)], out_hbm.at[f, d], osem)
            if p == 0:
                s1b, s2b = bucket(fb, B + 64)
                startsB = (jnp.int32(0), s1b, s2b, jnp.int32(B))
            elif p % 2 == 1:
                num_plane(p // 2)
        for oc in ocopies:
            if oc is not None:
                oc.wait()

        if ncopy[0] is not None:
            ncopy[0].wait()

    return k(tab_t, cat_t, ct_t, wb_flat)


def kernel(continuous, categorical, tables, W_num, b_num):
    tab_t = tables.transpose(0, 2, 1)
    cat_t = categorical.T
    ct_t = continuous.T
    wb_flat = jnp.concatenate([W_num.reshape(-1), b_num.reshape(-1)])
    out = _sc_embed(tab_t, cat_t, ct_t, wb_flat)
    return out.transpose(2, 0, 1)
